# baseline (speedup 1.0000x reference)
import functools

import jax
import jax.numpy as jnp
from jax import lax
from jax.experimental import pallas as pl
from jax.experimental.pallas import tpu as pltpu
from jax.experimental.pallas import tpu_sc as plsc

B, H, W, C = 4, 224, 224, 96
CH, CW = 7, 7
NBOX = 4000
NPAD = 4096
BOX_PER_TILE = 128
NG = BOX_PER_TILE // 16
NPIX = CH * CW
NENT = NPIX * 4
HALF = NENT // 2
OUTROW = NPIX * C
TBL = B * H * W
NC = 2


def _roialign_sc(images_flat, boxes_t):
    mesh = plsc.VectorSubcoreMesh(core_axis_name="c", subcore_axis_name="s")

    @functools.partial(
        pl.kernel,
        mesh=mesh,
        compiler_params=pltpu.CompilerParams(use_tc_tiling_on_sc=False, needs_layout_passes=False),
        out_type=jax.ShapeDtypeStruct((NBOX, OUTROW), jnp.float32),
        scratch_types=[
            pltpu.VMEM((16,), jnp.float32),
            pltpu.VMEM((16,), jnp.float32),
            pltpu.VMEM((16,), jnp.float32),
            pltpu.VMEM((16,), jnp.float32),
            pltpu.VMEM((CH * 16,), jnp.int32),
            pltpu.VMEM((CH * 16,), jnp.int32),
            pltpu.VMEM((CH * 16,), jnp.float32),
            pltpu.VMEM((CH * 16,), jnp.float32),
            pltpu.VMEM((CW * 16,), jnp.int32),
            pltpu.VMEM((CW * 16,), jnp.int32),
            pltpu.VMEM((CW * 16,), jnp.float32),
            pltpu.VMEM((CW * 16,), jnp.float32),
            pltpu.VMEM((BOX_PER_TILE * 2, HALF), jnp.int32),
            pltpu.VMEM((BOX_PER_TILE * NENT,), jnp.float32),
            pltpu.VMEM((NENT, C), jnp.float32),
            pltpu.VMEM((NENT, C), jnp.float32),
            pltpu.VMEM((OUTROW,), jnp.float32),
            pltpu.SemaphoreType.DMA,
            pltpu.SemaphoreType.DMA,
        ],
    )
    def k(tbl, boxt, out, y1b, x1b, y2b, x2b, tbs, bbs, yls, vys,
          lls, rrs, xls, vxs, idx2d, wts, rows0, rows1, stage, sem_a, sem_b):
        wid = lax.axis_index("s") * NC + lax.axis_index("c")
        base = wid * BOX_PER_TILE
        lanes = lax.iota(jnp.int32, 16)

        def group_body(g, _):
            gstart = base + g * 16
            pltpu.sync_copy(boxt.at[0, pl.ds(gstart, 16)], y1b)
            pltpu.sync_copy(boxt.at[1, pl.ds(gstart, 16)], x1b)
            pltpu.sync_copy(boxt.at[2, pl.ds(gstart, 16)], y2b)
            pltpu.sync_copy(boxt.at[3, pl.ds(gstart, 16)], x2b)
            y1 = y1b[...]
            x1 = x1b[...]
            y2 = y2b[...]
            x2 = x2b[...]
            hs = (y2 - y1) * float(H - 1) / float(CH - 1)
            ws_ = (x2 - x1) * float(W - 1) / float(CW - 1)
            bi = jnp.minimum((gstart + lanes) // 1000, B - 1)
            bibase = bi * (H * W)

            def k_body(kk, _):
                fk = kk.astype(jnp.float32)
                off = kk * 16 + lanes
                in_y = y1 * float(H - 1) + fk * hs
                vyf = jnp.where((in_y >= 0.0) & (in_y <= float(H - 1)), 1.0, 0.0)
                yc = jnp.clip(in_y, 0.0, float(H - 1))
                top = yc.astype(jnp.int32)
                bot = jnp.minimum(top + 1, H - 1)
                plsc.store_scatter(tbs, [off], bibase + top * W)
                plsc.store_scatter(bbs, [off], bibase + bot * W)
                plsc.store_scatter(yls, [off], yc - top.astype(jnp.float32))
                plsc.store_scatter(vys, [off], vyf)
                in_x = x1 * float(W - 1) + fk * ws_
                vxf = jnp.where((in_x >= 0.0) & (in_x <= float(W - 1)), 1.0, 0.0)
                xc = jnp.clip(in_x, 0.0, float(W - 1))
                lef = xc.astype(jnp.int32)
                rig = jnp.minimum(lef + 1, W - 1)
                plsc.store_scatter(lls, [off], lef)
                plsc.store_scatter(rrs, [off], rig)
                plsc.store_scatter(xls, [off], xc - lef.astype(jnp.float32))
                plsc.store_scatter(vxs, [off], vxf)
                return 0

            lax.fori_loop(0, CH, k_body, 0)
            lbv = g * 16 + lanes
            wbase = lbv * NENT
            rowb = lbv * 2

            def p_body(p, _):
                i = p // CW
                j = p - i * CW
                oy = i * 16 + lanes
                ox = j * 16 + lanes
                tb = plsc.load_gather(tbs, [oy])
                bb = plsc.load_gather(bbs, [oy])
                ylv = plsc.load_gather(yls, [oy])
                vyv = plsc.load_gather(vys, [oy])
                lv = plsc.load_gather(lls, [ox])
                rv = plsc.load_gather(rrs, [ox])
                xlv = plsc.load_gather(xls, [ox])
                vxv = plsc.load_gather(vxs, [ox])
                vv = vyv * vxv
                omy = 1.0 - ylv
                omx = 1.0 - xlv
                pairs = (
                    (tb + lv, omy * omx * vv),
                    (tb + rv, omy * xlv * vv),
                    (bb + lv, ylv * omx * vv),
                    (bb + rv, ylv * xlv * vv),
                )
                e0 = p * 4
                for cc, (iv, wv) in enumerate(pairs):
                    e = e0 + cc
                    er = e // HALF
                    ec = e - er * HALF
                    plsc.store_scatter(
                        idx2d, [rowb + er, jnp.full((16,), ec, jnp.int32)], iv)
                    plsc.store_scatter(wts, [wbase + e], wv)
                return 0

            lax.fori_loop(0, NPIX, p_body, 0)
            return 0

        lax.fori_loop(0, NG, group_body, 0)

        def fire(lb, rbuf, sem):
            row = lb * 2
            pltpu.async_copy(tbl.at[idx2d.at[row]], rbuf.at[pl.ds(0, HALF)], sem)
            pltpu.async_copy(tbl.at[idx2d.at[row + 1]], rbuf.at[pl.ds(HALF, HALF)], sem)

        def drain(rbuf, sem):
            pltpu.make_async_copy(tbl.at[idx2d.at[0]], rbuf.at[pl.ds(0, HALF)], sem).wait()
            pltpu.make_async_copy(tbl.at[idx2d.at[0]], rbuf.at[pl.ds(HALF, HALF)], sem).wait()

        def compute(lb, rbuf):
            gbox = base + lb

            @pl.when(gbox < NBOX)
            def _():
                wb = lb * NENT

                def pix(p, _):
                    w4 = wb + p * 4
                    wtl = plsc.load_gather(wts, [jnp.full((16,), w4, jnp.int32)])
                    wtr = plsc.load_gather(wts, [jnp.full((16,), w4 + 1, jnp.int32)])
                    wbl = plsc.load_gather(wts, [jnp.full((16,), w4 + 2, jnp.int32)])
                    wbr = plsc.load_gather(wts, [jnp.full((16,), w4 + 3, jnp.int32)])
                    r0 = p * 4
                    sb = p * C
                    for ch in range(C // 16):
                        col = ch * 16 + lanes
                        tlv = plsc.load_gather(rbuf, [jnp.full((16,), r0, jnp.int32), col])
                        trv = plsc.load_gather(rbuf, [jnp.full((16,), r0 + 1, jnp.int32), col])
                        blv = plsc.load_gather(rbuf, [jnp.full((16,), r0 + 2, jnp.int32), col])
                        brv = plsc.load_gather(rbuf, [jnp.full((16,), r0 + 3, jnp.int32), col])
                        acc = wtl * tlv + wtr * trv + wbl * blv + wbr * brv
                        plsc.store_scatter(stage, [sb + ch * 16 + lanes], acc)
                    return 0

                lax.fori_loop(0, NPIX, pix, 0)
                pltpu.sync_copy(stage, out.at[gbox])

        fire(0, rows0, sem_a)

        def ring(gq, _):
            lb0 = gq * 2
            fire(lb0 + 1, rows1, sem_b)
            drain(rows0, sem_a)
            compute(lb0, rows0)

            @pl.when(lb0 + 2 < BOX_PER_TILE)
            def _():
                fire(lb0 + 2, rows0, sem_a)

            drain(rows1, sem_b)
            compute(lb0 + 1, rows1)
            return 0

        lax.fori_loop(0, BOX_PER_TILE // 2, ring, 0)

    return k(images_flat, boxes_t)


def kernel(images, boxes):
    images_flat = images.reshape(TBL, C)
    flat_boxes = boxes.reshape(-1, 4)
    boxes_t = jnp.pad(flat_boxes, ((0, NPAD - NBOX), (0, 0))).T
    out = _roialign_sc(images_flat, boxes_t)
    return out.reshape(NBOX, CH, CW, C)
